# baseline (speedup 1.0000x reference)
import functools

import jax
import jax.numpy as jnp
from jax import lax
from jax.experimental import pallas as pl
from jax.experimental.pallas import tpu as pltpu
from jax.experimental.pallas import tpu_sc as plsc

B, R, N, L, D, H, Dh = 4, 8, 2048, 256, 3072, 24, 128
BR = B * R
D2 = 2 * D

E_BLK = 512
O_BLK = 512
N_BLK = 256
NB = N // N_BLK
NA = D2 // E_BLK
NO = D // O_BLK
NI = B * NB
NC, NS = 2, 16
LANES = 16


def _winner_sc_body(idx_hbm, mask_hbm, win_hbm, idx_v, win_v, rep_v, mask_v):
    wid = lax.axis_index("s") * NC + lax.axis_index("c")
    b = wid // NB
    n0 = (wid % NB) * N_BLK
    pltpu.sync_copy(idx_hbm.at[b], idx_v)
    pltpu.sync_copy(mask_hbm.at[b], mask_v)
    mv = mask_v[...]
    minus1 = jnp.full((LANES,), -1, jnp.int32)
    zeros = jnp.zeros((LANES,), jnp.int32)
    lane = lax.iota(jnp.int32, LANES)
    for c in range(N_BLK // LANES):
        win_v[0, pl.ds(c * LANES, LANES)] = minus1
    for j in range(R):
        valid = mv[j] == 1.0
        jvec = jnp.full((LANES,), j, jnp.int32)
        for c in range(L // LANES):
            idx = idx_v[j, pl.ds(c * LANES, LANES)]
            rel = idx - n0
            inb = (rel >= 0) & (rel < N_BLK)
            inb = jnp.logical_and(inb, valid)
            relc = jnp.clip(rel, 0, N_BLK - 1)
            plsc.store_scatter(win_v, [zeros, relc], jvec, mask=inb)
    for c in range(N_BLK // LANES):
        wchunk = win_v[0, pl.ds(c * LANES, LANES)]
        rowidx = c * LANES + lane
        for col in range(R):
            colv = jnp.full((LANES,), col, jnp.int32)
            plsc.store_scatter(rep_v, [rowidx, colv], wchunk)
    pltpu.sync_copy(rep_v, win_hbm.at[wid])


@functools.lru_cache(maxsize=1)
def _winner_sc():
    return pl.kernel(
        _winner_sc_body,
        out_type=jax.ShapeDtypeStruct((NI, N_BLK, R), jnp.int32),
        mesh=plsc.VectorSubcoreMesh(core_axis_name="c", subcore_axis_name="s"),
        compiler_params=pltpu.CompilerParams(needs_layout_passes=False),
        scratch_types=[
            pltpu.VMEM((R, L), jnp.int32),
            pltpu.VMEM((1, N_BLK), jnp.int32),
            pltpu.VMEM((N_BLK, R), jnp.int32),
            pltpu.VMEM((LANES,), jnp.float32),
        ],
    )


def _fused_body(x_ref, wlh_ref, blh_ref, alpha_ref, mask_ref, wout_ref,
                win_ref, img_ref, lo_ref, out_ref, vs_ref, ovec_ref):
    step = pl.program_id(0)

    @pl.when(step < NA)
    def _phase_a():
        kv = lax.dot_general(x_ref[...], wlh_ref[...],
                             (((1,), (1,)), ((), ())),
                             preferred_element_type=jnp.float32)
        kv = kv + blh_ref[...]
        valid = mask_ref[...] == 1.0
        lo_ref[...] = jnp.where(valid, kv, 0.0)

        @pl.when(step >= NA // 2)
        def _store_v():
            col = (step - NA // 2) * E_BLK
            vs_ref[:, pl.ds(col, E_BLK)] = kv * (1.0 - alpha_ref[...])

    @pl.when((step >= NA) & (step < NA + NO))
    def _phase_b():
        o = step - NA
        ov = lax.dot_general(vs_ref[...], wout_ref[...],
                             (((1,), (1,)), ((), ())),
                             preferred_element_type=jnp.float32)
        ovec_ref[:, pl.ds(o * O_BLK, O_BLK)] = ov

    @pl.when(step >= NA + NO)
    def _phase_c():
        i = step - (NA + NO)
        b = i // NB
        w8 = win_ref[0]
        jot = lax.broadcasted_iota(jnp.int32, (N_BLK, R), 1)
        oh = (w8 == jot).astype(jnp.float32)
        ov = ovec_ref[pl.ds(b * R, R), :]
        rows = lax.dot_general(oh, ov, (((1,), (0,)), ((), ())),
                               preferred_element_type=jnp.float32)
        out_ref[0] = jnp.where(w8[:, 0:1] >= 0, rows, img_ref[0])


def kernel(instance_tokens, image_tokens, img_idxs, layout_masks, alpha,
           W_lh, b_lh, W_q, W_out):
    x = instance_tokens.reshape(BR, D)
    alpha2 = alpha.reshape(BR, 1)
    mask2 = layout_masks.reshape(BR, 1)
    b_lh2 = b_lh.reshape(1, D2)
    mask16 = jnp.pad(layout_masks, ((0, 0), (0, LANES - R)))

    winner = _winner_sc()(img_idxs, mask16)

    G = NA + NO + NI

    def _ia(s):
        return jnp.minimum(s, NI - 1)

    lo, updated = pl.pallas_call(
        _fused_body,
        grid=(G,),
        in_specs=[
            pl.BlockSpec((BR, D), lambda s: (0, 0)),
            pl.BlockSpec((E_BLK, D), lambda s: (jnp.minimum(s, NA - 1), 0)),
            pl.BlockSpec((1, E_BLK), lambda s: (0, jnp.minimum(s, NA - 1))),
            pl.BlockSpec((BR, 1), lambda s: (0, 0)),
            pl.BlockSpec((BR, 1), lambda s: (0, 0)),
            pl.BlockSpec((O_BLK, D),
                         lambda s: (jnp.clip(s - NA, 0, NO - 1), 0)),
            pl.BlockSpec((1, N_BLK, R),
                         lambda s: (jnp.clip(s - NA - NO, 0, NI - 1), 0, 0)),
            pl.BlockSpec((1, N_BLK, D),
                         lambda s: (jnp.clip(s - NA - NO, 0, NI - 1) // NB,
                                    jnp.clip(s - NA - NO, 0, NI - 1) % NB,
                                    0)),
        ],
        out_specs=[
            pl.BlockSpec((BR, E_BLK), lambda s: (0, jnp.minimum(s, NA - 1))),
            pl.BlockSpec((1, N_BLK, D),
                         lambda s: (jnp.clip(s - NA - NO, 0, NI - 1) // NB,
                                    jnp.clip(s - NA - NO, 0, NI - 1) % NB,
                                    0)),
        ],
        out_shape=[
            jax.ShapeDtypeStruct((BR, D2), jnp.float32),
            jax.ShapeDtypeStruct((B, N, D), jnp.float32),
        ],
        scratch_shapes=[
            pltpu.VMEM((BR, D), jnp.float32),
            pltpu.VMEM((BR, D), jnp.float32),
        ],
    )(x, W_lh, b_lh2, alpha2, mask2, W_out, winner, image_tokens)

    layout_outputs = lo.reshape(B, R, D2)
    return updated, layout_outputs


# speedup vs baseline: 1.0730x; 1.0730x over previous
import functools

import jax
import jax.numpy as jnp
from jax import lax
from jax.experimental import pallas as pl
from jax.experimental.pallas import tpu as pltpu
from jax.experimental.pallas import tpu_sc as plsc

B, R, N, L, D, H, Dh = 4, 8, 2048, 256, 3072, 24, 128
BR = B * R
D2 = 2 * D

E_BLK = 512
O_BLK = 512
NA = D2 // E_BLK
NO = D // O_BLK
T_SC = 256
NT = B * (N // T_SC)
N_BLK = 512
NB = N // N_BLK
NC, NS = 2, 16
LANES = 16


def _winner_sc_body(idx_hbm, mask_hbm, win_hbm, idx_v, win_v, rep_v, mask_v):
    wid = lax.axis_index("s") * NC + lax.axis_index("c")
    nb_per_b = N // T_SC
    b = wid // nb_per_b
    n0 = (wid % nb_per_b) * T_SC
    pltpu.sync_copy(idx_hbm.at[b], idx_v)
    pltpu.sync_copy(mask_hbm, mask_v)
    minus1 = jnp.full((LANES,), -1, jnp.int32)
    zeros = jnp.zeros((LANES,), jnp.int32)
    lane = lax.iota(jnp.int32, LANES)
    bvec = jnp.full((LANES,), b, jnp.int32)
    for c in range(T_SC // LANES):
        win_v[0, pl.ds(c * LANES, LANES)] = minus1
    for j in range(R):
        mval = plsc.load_gather(mask_v, [bvec, jnp.full((LANES,), j, jnp.int32)])
        validv = mval == 1.0
        jvec = jnp.full((LANES,), j, jnp.int32)
        for c in range(L // LANES):
            idx = idx_v[j, pl.ds(c * LANES, LANES)]
            rel = idx - n0
            inb = (rel >= 0) & (rel < T_SC)
            inb = jnp.logical_and(inb, validv)
            relc = jnp.clip(rel, 0, T_SC - 1)
            plsc.store_scatter(win_v, [zeros, relc], jvec, mask=inb)
    for c in range(T_SC // LANES):
        wchunk = win_v[0, pl.ds(c * LANES, LANES)]
        rowidx = c * LANES + lane
        for col in range(R):
            colv = jnp.full((LANES,), col, jnp.int32)
            plsc.store_scatter(rep_v, [rowidx, colv], wchunk)
    pltpu.sync_copy(rep_v, win_hbm.at[b, pl.ds(n0, T_SC), :])


@functools.lru_cache(maxsize=1)
def _winner_sc():
    return pl.kernel(
        _winner_sc_body,
        out_type=jax.ShapeDtypeStruct((B, N, R), jnp.int32),
        mesh=plsc.VectorSubcoreMesh(core_axis_name="c", subcore_axis_name="s"),
        compiler_params=pltpu.CompilerParams(needs_layout_passes=False),
        scratch_types=[
            pltpu.VMEM((R, L), jnp.int32),
            pltpu.VMEM((1, T_SC), jnp.int32),
            pltpu.VMEM((T_SC, R), jnp.int32),
            pltpu.VMEM((B, LANES), jnp.float32),
        ],
    )


def _matmul_body(x_ref, wlh_ref, blh_ref, alpha_ref, mask_ref, wout_ref,
                 lo_ref, ov_ref, vs_ref):
    step = pl.program_id(0)

    @pl.when(step < NA)
    def _phase_a():
        x2 = x_ref[...].reshape(BR, D)
        kv = lax.dot_general(x2, wlh_ref[...], (((1,), (1,)), ((), ())),
                             preferred_element_type=jnp.float32)
        kv = kv + blh_ref[...]
        kv3 = kv.reshape(B, R, E_BLK)
        valid = mask_ref[...][..., None] == 1.0
        lo_ref[...] = jnp.where(valid, kv3, 0.0)

        @pl.when(step >= NA // 2)
        def _store_v():
            col = (step - NA // 2) * E_BLK
            scale = 1.0 - alpha_ref[...][..., None]
            vs_ref[:, pl.ds(col, E_BLK)] = (kv3 * scale).reshape(BR, E_BLK)

    @pl.when(step >= NA)
    def _phase_b():
        ov_ref[...] = lax.dot_general(vs_ref[...], wout_ref[...],
                                      (((1,), (1,)), ((), ())),
                                      preferred_element_type=jnp.float32)


def _assemble_body(img_ref, win_ref, ov_ref, out_ref):
    w8 = win_ref[0]
    jot = lax.broadcasted_iota(jnp.int32, (N_BLK, R), 1)
    oh = (w8 == jot).astype(jnp.float32)
    rows = lax.dot_general(oh, ov_ref[...], (((1,), (0,)), ((), ())),
                           preferred_element_type=jnp.float32)
    out_ref[0] = jnp.where(w8[:, 0:1] >= 0, rows, img_ref[0])


def kernel(instance_tokens, image_tokens, img_idxs, layout_masks, alpha,
           W_lh, b_lh, W_q, W_out):
    b_lh2 = b_lh.reshape(1, D2)
    mask16 = jnp.pad(layout_masks, ((0, 0), (0, LANES - R)))

    winner = _winner_sc()(img_idxs, mask16)

    lo, out_vec = pl.pallas_call(
        _matmul_body,
        grid=(NA + NO,),
        in_specs=[
            pl.BlockSpec((B, R, D), lambda s: (0, 0, 0)),
            pl.BlockSpec((E_BLK, D), lambda s: (jnp.minimum(s, NA - 1), 0)),
            pl.BlockSpec((1, E_BLK), lambda s: (0, jnp.minimum(s, NA - 1))),
            pl.BlockSpec((B, R), lambda s: (0, 0)),
            pl.BlockSpec((B, R), lambda s: (0, 0)),
            pl.BlockSpec((O_BLK, D),
                         lambda s: (jnp.clip(s - NA, 0, NO - 1), 0)),
        ],
        out_specs=[
            pl.BlockSpec((B, R, E_BLK),
                         lambda s: (0, 0, jnp.minimum(s, NA - 1))),
            pl.BlockSpec((BR, O_BLK),
                         lambda s: (0, jnp.clip(s - NA, 0, NO - 1))),
        ],
        out_shape=[
            jax.ShapeDtypeStruct((B, R, D2), jnp.float32),
            jax.ShapeDtypeStruct((BR, D), jnp.float32),
        ],
        scratch_shapes=[pltpu.VMEM((BR, D), jnp.float32)],
    )(instance_tokens, W_lh, b_lh2, alpha, layout_masks, W_out)

    updated = pl.pallas_call(
        _assemble_body,
        grid=(B * NB,),
        in_specs=[
            pl.BlockSpec((1, N_BLK, D), lambda i: (i // NB, i % NB, 0)),
            pl.BlockSpec((1, N_BLK, R), lambda i: (i // NB, i % NB, 0)),
            pl.BlockSpec((R, D), lambda i: (i // NB, 0)),
        ],
        out_specs=pl.BlockSpec((1, N_BLK, D), lambda i: (i // NB, i % NB, 0)),
        out_shape=jax.ShapeDtypeStruct((B, N, D), jnp.float32),
    )(image_tokens, winner, out_vec)

    return updated, lo


# speedup vs baseline: 1.0867x; 1.0128x over previous
import functools

import jax
import jax.numpy as jnp
from jax import lax
from jax.experimental import pallas as pl
from jax.experimental.pallas import tpu as pltpu
from jax.experimental.pallas import tpu_sc as plsc

B, R, N, L, D, H, Dh = 4, 8, 2048, 256, 3072, 24, 128
BR = B * R
D2 = 2 * D

E_BLK = 512
O_BLK = 512
NA = D2 // E_BLK
NO = D // O_BLK
T_SC = 256
NT = B * (N // T_SC)
N_BLK = 1024
NB = N // N_BLK
NC, NS = 2, 16
LANES = 16


def _winner_sc_body(idx_hbm, mask_hbm, win_hbm, idx_v, win_v, rep_v, mask_v):
    wid = lax.axis_index("s") * NC + lax.axis_index("c")
    nb_per_b = N // T_SC
    b = wid // nb_per_b
    n0 = (wid % nb_per_b) * T_SC
    pltpu.sync_copy(idx_hbm.at[b], idx_v)
    pltpu.sync_copy(mask_hbm, mask_v)
    minus1 = jnp.full((LANES,), -1, jnp.int32)
    zeros = jnp.zeros((LANES,), jnp.int32)
    lane = lax.iota(jnp.int32, LANES)
    bvec = jnp.full((LANES,), b, jnp.int32)
    for c in range(T_SC // LANES):
        win_v[0, pl.ds(c * LANES, LANES)] = minus1
    for j in range(R):
        mval = plsc.load_gather(mask_v, [bvec, jnp.full((LANES,), j, jnp.int32)])
        validv = mval == 1.0
        jvec = jnp.full((LANES,), j, jnp.int32)
        for c in range(L // LANES):
            idx = idx_v[j, pl.ds(c * LANES, LANES)]
            rel = idx - n0
            inb = (rel >= 0) & (rel < T_SC)
            inb = jnp.logical_and(inb, validv)
            relc = jnp.clip(rel, 0, T_SC - 1)
            plsc.store_scatter(win_v, [zeros, relc], jvec, mask=inb)
    for c in range(T_SC // LANES):
        wchunk = win_v[0, pl.ds(c * LANES, LANES)]
        rowidx = c * LANES + lane
        for col in range(R):
            colv = jnp.full((LANES,), col, jnp.int32)
            plsc.store_scatter(rep_v, [rowidx, colv], wchunk)
    pltpu.sync_copy(rep_v, win_hbm.at[b, pl.ds(n0, T_SC), :])


@functools.lru_cache(maxsize=1)
def _winner_sc():
    return pl.kernel(
        _winner_sc_body,
        out_type=jax.ShapeDtypeStruct((B, N, R), jnp.int32),
        mesh=plsc.VectorSubcoreMesh(core_axis_name="c", subcore_axis_name="s"),
        compiler_params=pltpu.CompilerParams(needs_layout_passes=False),
        scratch_types=[
            pltpu.VMEM((R, L), jnp.int32),
            pltpu.VMEM((1, T_SC), jnp.int32),
            pltpu.VMEM((T_SC, R), jnp.int32),
            pltpu.VMEM((B, R), jnp.float32),
        ],
    )


def _matmul_body(x_ref, wlh_ref, blh_ref, alpha_ref, mask_ref, wout_ref,
                 lo_ref, ov_ref, vs_ref):
    step = pl.program_id(0)

    @pl.when(step < NA)
    def _phase_a():
        x2 = x_ref[...].reshape(BR, D)
        kv = lax.dot_general(x2, wlh_ref[...], (((1,), (1,)), ((), ())),
                             preferred_element_type=jnp.float32)
        kv = kv + blh_ref[...]
        kv3 = kv.reshape(B, R, E_BLK)
        valid = mask_ref[...][..., None] == 1.0
        lo_ref[...] = jnp.where(valid, kv3, 0.0)

        @pl.when(step >= NA // 2)
        def _store_v():
            col = (step - NA // 2) * E_BLK
            scale = 1.0 - alpha_ref[...][..., None]
            vs_ref[:, pl.ds(col, E_BLK)] = (kv3 * scale).reshape(BR, E_BLK)

    @pl.when(step >= NA)
    def _phase_b():
        ov_ref[...] = lax.dot_general(vs_ref[...], wout_ref[...],
                                      (((1,), (1,)), ((), ())),
                                      preferred_element_type=jnp.float32)


def _assemble_body(img_ref, win_ref, ov_ref, out_ref):
    w8 = win_ref[0]
    jot = lax.broadcasted_iota(jnp.int32, (N_BLK, R), 1)
    oh = (w8 == jot).astype(jnp.float32)
    rows = lax.dot_general(oh, ov_ref[...], (((1,), (0,)), ((), ())),
                           preferred_element_type=jnp.float32)
    out_ref[0] = jnp.where(w8[:, 0:1] >= 0, rows, img_ref[0])


def kernel(instance_tokens, image_tokens, img_idxs, layout_masks, alpha,
           W_lh, b_lh, W_q, W_out):
    b_lh2 = b_lh.reshape(1, D2)

    winner = _winner_sc()(img_idxs, layout_masks)

    lo, out_vec = pl.pallas_call(
        _matmul_body,
        grid=(NA + NO,),
        in_specs=[
            pl.BlockSpec((B, R, D), lambda s: (0, 0, 0)),
            pl.BlockSpec((E_BLK, D), lambda s: (jnp.minimum(s, NA - 1), 0)),
            pl.BlockSpec((1, E_BLK), lambda s: (0, jnp.minimum(s, NA - 1))),
            pl.BlockSpec((B, R), lambda s: (0, 0)),
            pl.BlockSpec((B, R), lambda s: (0, 0)),
            pl.BlockSpec((O_BLK, D),
                         lambda s: (jnp.clip(s - NA, 0, NO - 1), 0)),
        ],
        out_specs=[
            pl.BlockSpec((B, R, E_BLK),
                         lambda s: (0, 0, jnp.minimum(s, NA - 1))),
            pl.BlockSpec((BR, O_BLK),
                         lambda s: (0, jnp.clip(s - NA, 0, NO - 1))),
        ],
        out_shape=[
            jax.ShapeDtypeStruct((B, R, D2), jnp.float32),
            jax.ShapeDtypeStruct((BR, D), jnp.float32),
        ],
        scratch_shapes=[pltpu.VMEM((BR, D), jnp.float32)],
    )(instance_tokens, W_lh, b_lh2, alpha, layout_masks, W_out)

    updated = pl.pallas_call(
        _assemble_body,
        grid=(B * NB,),
        in_specs=[
            pl.BlockSpec((1, N_BLK, D), lambda i: (i // NB, i % NB, 0)),
            pl.BlockSpec((1, N_BLK, R), lambda i: (i // NB, i % NB, 0)),
            pl.BlockSpec((R, D), lambda i: (i // NB, 0)),
        ],
        out_specs=pl.BlockSpec((1, N_BLK, D), lambda i: (i // NB, i % NB, 0)),
        out_shape=jax.ShapeDtypeStruct((B, N, D), jnp.float32),
    )(image_tokens, winner, out_vec)

    return updated, lo
